# baseline (speedup 1.0000x reference)
import jax
import jax.numpy as jnp
from jax.experimental import pallas as pl
from jax.experimental.pallas import tpu as pltpu

_K = 4
_L = 4
_CH = 128

_UT, _D, _MCOL = 0, 1, 2


def _gelu(x):
    return 0.5 * x * (1.0 + jnp.tanh(0.7978845608028654 * (x + 0.044715 * x * x * x)))


def _answer_finder_kernel(x_ref, mr_ref, W0_ref, b0_ref, w1_ref,
                          W2_ref, b2_ref, w3_ref, out_ref,
                          h_s, cols_s, arow_s):
    k = pl.program_id(1)
    s = _K * _CH

    @pl.when(k < _K)
    def _phase1():
        xk = x_ref[0]
        hk = _gelu(jnp.dot(xk, W0_ref[...],
                           preferred_element_type=jnp.float32) + b0_ref[...])
        h_s[pl.ds(k * _CH, _CH), :] = hk

    @pl.when(k == _K - 1)
    def _stats():
        h = h_s[...]
        w1 = w1_ref[...]
        w3 = w3_ref[...]
        mrowf = mr_ref[0].astype(jnp.float32)
        mrowb = mrowf > 0.0

        cdims = (((1,), (1,)), ((), ()))
        sl = jax.lax.dot_general(w1, h, cdims,
                                 preferred_element_type=jnp.float32)
        a_row = jax.lax.dot_general(w3, h, cdims,
                                    preferred_element_type=jnp.float32)
        sc = _gelu(jnp.dot(h, W2_ref[...],
                           preferred_element_type=jnp.float32) + b2_ref[...])
        c_row = jax.lax.dot_general(w3, sc, cdims,
                                    preferred_element_type=jnp.float32)

        slm = mrowf * sl + (mrowf - 1.0) * 10.0
        m1 = jnp.max(slm)
        z1 = jnp.sum(jnp.exp(slm - m1))
        slp = (m1 + jnp.log(z1)) - slm

        neg = jnp.float32(-1e30)
        ma = jnp.max(jnp.where(mrowb, a_row, neg))
        mc = jnp.max(jnp.where(mrowb, c_row, neg))
        m2 = jnp.maximum(ma + mc, -10.0)
        ea = jnp.where(mrowb, jnp.exp(a_row - ma), 0.0)
        ec = jnp.where(mrowb, jnp.exp(c_row - mc), 0.0)

        pad = jnp.zeros_like(mrowf)
        stack = jnp.concatenate(
            [mrowf, ea, ec, slp, c_row, pad, pad, pad], axis=0)
        colsT = jnp.transpose(stack, (1, 0))
        mcolf = colsT[:, 0:1]
        ea_c = colsT[:, 1:2]
        ec_c = colsT[:, 2:3]
        slp_c = colsT[:, 3:4]
        c_c = colsT[:, 4:5]

        ii = jax.lax.broadcasted_iota(jnp.int32, (s, s), 0)
        jj = jax.lax.broadcasted_iota(jnp.int32, (s, s), 1)
        tri_f = jnp.where(jj >= ii, 1.0, 0.0)
        rhs = jnp.concatenate([ea_c, mcolf], axis=1)
        suf = jax.lax.dot_general(tri_f, rhs, (((1,), (0,)), ((), ())),
                                  preferred_element_type=jnp.float32)
        z2p = jnp.sum(suf[:, 0:1] * ec_c)
        npairs = jnp.sum(suf[:, 1:2] * mcolf)
        z2 = z2p * jnp.exp((ma + mc) - m2) \
            + (s * s - npairs) * jnp.exp(-10.0 - m2)
        lse2 = m2 + jnp.log(z2)

        arow_s[...] = a_row
        cols_s[:, _UT:_UT + 1] = slp_c + (lse2 + 10.0)
        cols_s[:, _D:_D + 1] = c_c + 10.0
        cols_s[:, _MCOL:_MCOL + 1] = mcolf

    @pl.when(k >= _K)
    def _phase2():
        r0 = (k - _K) * _CH
        ut = cols_s[pl.ds(r0, _CH), _UT:_UT + 1]
        d = cols_s[pl.ds(r0, _CH), _D:_D + 1]
        mcol = cols_s[pl.ds(r0, _CH), _MCOL:_MCOL + 1]
        mrowf = mr_ref[0].astype(jnp.float32)
        a_row = arow_s[...]

        ii = jax.lax.broadcasted_iota(jnp.int32, (_CH, s), 0) + r0
        jj = jax.lax.broadcasted_iota(jnp.int32, (_CH, s), 1)
        valid_f = jnp.where(jj >= ii, 1.0, 0.0) * (mcol * mrowf)
        out_ref[0] = ut - valid_f * (d + a_row)


@jax.jit
def kernel(inputs, mask, W0, b0, w1, W2, b2, w3):
    B, S, D = inputs.shape
    U = W0.shape[1]
    mr = mask.reshape(B, 1, S)
    grid = (B, _K + _L)
    in_specs = [
            pl.BlockSpec((1, _CH, D),
                         lambda b, k: (b, jnp.minimum(k, _K - 1), 0)),
            pl.BlockSpec((1, 1, S), lambda b, k: (b, 0, 0)),
            pl.BlockSpec((D, U), lambda b, k: (0, 0)),
            pl.BlockSpec((1, U), lambda b, k: (0, 0)),
            pl.BlockSpec((1, U), lambda b, k: (0, 0)),
            pl.BlockSpec((U, U), lambda b, k: (0, 0)),
            pl.BlockSpec((1, U), lambda b, k: (0, 0)),
            pl.BlockSpec((1, U), lambda b, k: (0, 0)),
    ]
    out_specs = pl.BlockSpec((1, _CH, S),
                             lambda b, k: (b, jnp.maximum(k - _K, 0), 0))
    return pl.pallas_call(
        _answer_finder_kernel,
        grid=grid,
        in_specs=in_specs,
        out_specs=out_specs,
        out_shape=jax.ShapeDtypeStruct((B, S, S), jnp.float32),
        scratch_shapes=[
            pltpu.VMEM((S, U), jnp.float32),
            pltpu.VMEM((S, 8), jnp.float32),
            pltpu.VMEM((1, S), jnp.float32),
        ],
    )(inputs, mr, W0, b0.reshape(1, U), w1.reshape(1, U),
      W2, b2.reshape(1, U), w3.reshape(1, U))


# speedup vs baseline: 1.7237x; 1.7237x over previous
import jax
import jax.numpy as jnp
from jax.experimental import pallas as pl
from jax.experimental.pallas import tpu as pltpu


def _gelu(x):
    return 0.5 * x * (1.0 + jnp.tanh(0.7978845608028654 * (x + 0.044715 * x * x * x)))


def _suffix_sum(row, s):
    padded = jnp.concatenate([row, jnp.zeros_like(row)], axis=1)
    d = 1
    while d < s:
        padded = padded + pltpu.roll(padded, 2 * s - d, 1)
        d *= 2
    return padded[:, :s]


def _answer_finder_kernel(x_ref, mr_ref, W0_ref, b0_ref, w1_ref,
                          W2_ref, b2_ref, w3_ref, out_ref):
    x = x_ref[0]
    s = out_ref.shape[1]
    w1 = w1_ref[...]
    w3 = w3_ref[...]
    mrowf = mr_ref[0].astype(jnp.float32)
    mrowb = mrowf > 0.0

    h = _gelu(jnp.dot(x, W0_ref[...],
                      preferred_element_type=jnp.float32) + b0_ref[...])

    cdims = (((1,), (1,)), ((), ()))
    sl = jax.lax.dot_general(w1, h, cdims, preferred_element_type=jnp.float32)
    a_row = jax.lax.dot_general(w3, h, cdims,
                                preferred_element_type=jnp.float32)
    sc = _gelu(jnp.dot(h, W2_ref[...],
                       preferred_element_type=jnp.float32) + b2_ref[...])
    c_row = jax.lax.dot_general(w3, sc, cdims,
                                preferred_element_type=jnp.float32)

    slm = mrowf * sl + (mrowf - 1.0) * 10.0
    m1 = jnp.max(slm)
    z1 = jnp.sum(jnp.exp(slm - m1))
    slp = (m1 + jnp.log(z1)) - slm

    neg = jnp.float32(-1e30)
    ma = jnp.max(jnp.where(mrowb, a_row, neg))
    mc = jnp.max(jnp.where(mrowb, c_row, neg))
    m2 = jnp.maximum(ma + mc, -10.0)
    ea = jnp.where(mrowb, jnp.exp(a_row - ma), 0.0)
    ec = jnp.where(mrowb, jnp.exp(c_row - mc), 0.0)
    sa = _suffix_sum(ea, s)
    cnt = _suffix_sum(mrowf, s)
    z2p = jnp.sum(ec * sa)
    npairs = jnp.sum(mrowf * cnt)
    z2 = z2p * jnp.exp((ma + mc) - m2) \
        + (s * s - npairs) * jnp.exp(-10.0 - m2)
    lse2 = m2 + jnp.log(z2)

    ut_row = slp + (lse2 + 10.0)
    d_row = c_row + 10.0

    pad = jnp.zeros_like(mrowf)
    stack = jnp.concatenate(
        [ut_row, d_row, mrowf, pad, pad, pad, pad, pad], axis=0)
    colsT = jnp.transpose(stack, (1, 0))
    ut_c = colsT[:, 0:1]
    d_c = colsT[:, 1:2]
    mcolb = colsT[:, 2:3] > 0.0

    ii = jax.lax.broadcasted_iota(jnp.int32, (s, s), 0)
    jj = jax.lax.broadcasted_iota(jnp.int32, (s, s), 1)
    vb = (jj >= ii) & (mcolb & mrowb)
    out_ref[0] = ut_c - jnp.where(vb, d_c + a_row, 0.0)


@jax.jit
def kernel(inputs, mask, W0, b0, w1, W2, b2, w3):
    B, S, D = inputs.shape
    U = W0.shape[1]
    mr = mask.reshape(B, 1, S)
    return pl.pallas_call(
        _answer_finder_kernel,
        grid=(B,),
        in_specs=[
            pl.BlockSpec((1, S, D), lambda b: (b, 0, 0)),
            pl.BlockSpec((1, 1, S), lambda b: (b, 0, 0)),
            pl.BlockSpec((D, U), lambda b: (0, 0)),
            pl.BlockSpec((1, U), lambda b: (0, 0)),
            pl.BlockSpec((1, U), lambda b: (0, 0)),
            pl.BlockSpec((U, U), lambda b: (0, 0)),
            pl.BlockSpec((1, U), lambda b: (0, 0)),
            pl.BlockSpec((1, U), lambda b: (0, 0)),
        ],
        out_specs=pl.BlockSpec((1, S, S), lambda b: (b, 0, 0)),
        out_shape=jax.ShapeDtypeStruct((B, S, S), jnp.float32),
    )(inputs, mr, W0, b0.reshape(1, U), w1.reshape(1, U),
      W2, b2.reshape(1, U), w3.reshape(1, U))


# speedup vs baseline: 1.8841x; 1.0931x over previous
import jax
import jax.numpy as jnp
from jax.experimental import pallas as pl
from jax.experimental.pallas import tpu as pltpu


def _gelu(x):
    return 0.5 * x * (1.0 + jnp.tanh(0.7978845608028654 * (x + 0.044715 * x * x * x)))


def _suffix_sum(row, s):
    padded = jnp.concatenate([row, jnp.zeros_like(row)], axis=1)
    d = 1
    while d < s:
        padded = padded + pltpu.roll(padded, 2 * s - d, 1)
        d *= 2
    return padded[:, :s]


def _answer_finder_kernel(x_ref, mr_ref, W0_ref, b0_ref, w1_ref,
                          W2_ref, b2_ref, w3_ref, out_ref):
    x = x_ref[0]
    s = out_ref.shape[1]
    w1 = w1_ref[...]
    w3 = w3_ref[...]
    mrowf = mr_ref[0].astype(jnp.float32)
    mrowb = mrowf > 0.0

    h = _gelu(jnp.dot(x, W0_ref[...],
                      preferred_element_type=jnp.float32) + b0_ref[...])

    cdims = (((1,), (1,)), ((), ()))
    sl = jax.lax.dot_general(w1, h, cdims, preferred_element_type=jnp.float32)
    a_row = jax.lax.dot_general(w3, h, cdims,
                                preferred_element_type=jnp.float32)
    sc = _gelu(jnp.dot(h, W2_ref[...],
                       preferred_element_type=jnp.float32) + b2_ref[...])
    c_row = jax.lax.dot_general(w3, sc, cdims,
                                preferred_element_type=jnp.float32)

    slm = mrowf * sl + (mrowf - 1.0) * 10.0
    m1 = jnp.max(slm)
    z1 = jnp.sum(jnp.exp(slm - m1))
    slp = (m1 + jnp.log(z1)) - slm

    neg = jnp.float32(-1e30)
    ma = jnp.max(jnp.where(mrowb, a_row, neg))
    mc = jnp.max(jnp.where(mrowb, c_row, neg))
    m2 = jnp.maximum(ma + mc, -10.0)
    ea = jnp.where(mrowb, jnp.exp(a_row - ma), 0.0)
    ec = jnp.where(mrowb, jnp.exp(c_row - mc), 0.0)
    sa = _suffix_sum(ea, s)
    z2p = jnp.sum(ec * sa)
    p = jnp.sum(mrowf)
    npairs = 0.5 * p * (p + 1.0)
    z2 = z2p * jnp.exp((ma + mc) - m2) \
        + (s * s - npairs) * jnp.exp(-10.0 - m2)
    lse2 = m2 + jnp.log(z2)

    ut_row = slp + (lse2 + 10.0)
    d_row = c_row + 10.0

    pad = jnp.zeros_like(mrowf)
    stack = jnp.concatenate(
        [ut_row, d_row, mrowf, pad, pad, pad, pad, pad], axis=0)
    colsT = jnp.transpose(stack, (1, 0))
    ut_c = colsT[:, 0:1]
    d_c = colsT[:, 1:2]
    mcolb = colsT[:, 2:3] > 0.0

    ii = jax.lax.broadcasted_iota(jnp.int32, (s, s), 0)
    jj = jax.lax.broadcasted_iota(jnp.int32, (s, s), 1)
    vb = (jj >= ii) & (mcolb & mrowb)
    out_ref[0] = ut_c - jnp.where(vb, d_c + a_row, 0.0)


@jax.jit
def kernel(inputs, mask, W0, b0, w1, W2, b2, w3):
    B, S, D = inputs.shape
    U = W0.shape[1]
    mr = mask.reshape(B, 1, S)
    return pl.pallas_call(
        _answer_finder_kernel,
        grid=(B,),
        in_specs=[
            pl.BlockSpec((1, S, D), lambda b: (b, 0, 0)),
            pl.BlockSpec((1, 1, S), lambda b: (b, 0, 0)),
            pl.BlockSpec((D, U), lambda b: (0, 0)),
            pl.BlockSpec((1, U), lambda b: (0, 0)),
            pl.BlockSpec((1, U), lambda b: (0, 0)),
            pl.BlockSpec((U, U), lambda b: (0, 0)),
            pl.BlockSpec((1, U), lambda b: (0, 0)),
            pl.BlockSpec((1, U), lambda b: (0, 0)),
        ],
        out_specs=pl.BlockSpec((1, S, S), lambda b: (b, 0, 0)),
        out_shape=jax.ShapeDtypeStruct((B, S, S), jnp.float32),
    )(inputs, mr, W0, b0.reshape(1, U), w1.reshape(1, U),
      W2, b2.reshape(1, U), w3.reshape(1, U))
